# Initial kernel scaffold; baseline (speedup 1.0000x reference)
import jax
import jax.numpy as jnp
from jax.experimental import pallas as pl


def kernel(x_bus, x_generator, ei_bb, ei_bg, ei_gb, lin_bus_W, lin_bus_b, lin_gen_W, lin_gen_b, c0_bb_Wl, c0_bb_bl, c0_bb_Wr, c0_bg_Wl, c0_bg_bl, c0_bg_Wr, c0_gb_Wl, c0_gb_bl, c0_gb_Wr, c1_bb_Wl, c1_bb_bl, c1_bb_Wr, c1_bg_Wl, c1_bg_bl, c1_bg_Wr, c1_gb_Wl, c1_gb_bl, c1_gb_Wr, out_bus_W, out_bus_b, out_gen_W, out_gen_b):
    raise NotImplementedError("write your pallas kernel here")


# speedup vs baseline: 3.0025x; 3.0025x over previous
import functools

import jax
import jax.numpy as jnp
from jax import lax
from jax.experimental import pallas as pl
from jax.experimental.pallas import tpu as pltpu
from jax.experimental.pallas import tpu_sc as plsc

N = 10000
E = 160000
DIN = 256
H = 512
OUT = 2

KC = 4
CW = 128
BM = 1000
SUB = 16
EPW = E // SUB
TIL = 128
NT = (EPW + TIL - 1) // TIL
EPAD = NT * TIL
ACC = 10240
DEAD = 10200
ZR = ACC // SUB
WRS = 624
WRN = 640


def _proj_body(x_ref, w_ref, b_ref, o_ref):
    t = jnp.dot(x_ref[...], w_ref[...], preferred_element_type=jnp.float32)
    t = jnp.maximum(t + b_ref[...], 0.0)
    for kc in range(KC):
        o_ref[kc] = t[:, kc * CW:(kc + 1) * CW]


def _proj(x, wt, b):
    return pl.pallas_call(
        _proj_body,
        grid=(N // BM,),
        in_specs=[
            pl.BlockSpec((BM, DIN), lambda m: (m, 0)),
            pl.BlockSpec((DIN, H), lambda m: (0, 0)),
            pl.BlockSpec((1, H), lambda m: (0, 0)),
        ],
        out_specs=pl.BlockSpec((KC, BM, CW), lambda m: (0, m, 0)),
        out_shape=jax.ShapeDtypeStruct((KC, N, CW), jnp.float32),
    )(x, wt, b)


def _make_mix_body(n_rel, do_proj, relu_col):
    def body(*refs):
        i = 0
        s = refs[i:i + n_rel]; i += n_rel
        c = refs[i:i + n_rel]; i += n_rel
        x_ref = refs[i]; i += 1
        w_ref = refs[i]; i += 1
        b_ref = refs[i]; i += 1
        if do_proj:
            wo_ref = refs[i]; i += 1
            bo_ref = refs[i]; i += 1
        o_ref = refs[i]
        parts = []
        for k in range(n_rel):
            r = 1.0 / jnp.maximum(c[k][...], 1.0)
            for kc in range(KC):
                parts.append(s[k][kc] * r)
        for kc in range(KC):
            parts.append(x_ref[kc])
        z = jnp.concatenate(parts, axis=1)
        t = jnp.dot(z, w_ref[...], preferred_element_type=jnp.float32) + b_ref[...]
        if do_proj:
            o = jnp.dot(t, wo_ref[...], preferred_element_type=jnp.float32) + bo_ref[...]
            if relu_col is not None:
                col = lax.broadcasted_iota(jnp.int32, o.shape, 1)
                o = jnp.where(col == relu_col, jnp.maximum(o, 0.0), o)
            o_ref[...] = o
        else:
            for kc in range(KC):
                o_ref[kc] = t[:, kc * CW:(kc + 1) * CW]
    return body


def _mix(n_rel, do_proj, relu_col, ss, cs, x, wcat, bias, wo=None, bo=None):
    kz = (n_rel + 1) * H
    in_specs = []
    args = []
    for s in ss:
        in_specs.append(pl.BlockSpec((KC, BM, CW), lambda m: (0, m, 0)))
        args.append(s)
    for c in cs:
        in_specs.append(pl.BlockSpec((BM, 1), lambda m: (m, 0)))
        args.append(c)
    in_specs.append(pl.BlockSpec((KC, BM, CW), lambda m: (0, m, 0)))
    args.append(x)
    in_specs.append(pl.BlockSpec((kz, H), lambda m: (0, 0)))
    args.append(wcat)
    in_specs.append(pl.BlockSpec((1, H), lambda m: (0, 0)))
    args.append(bias)
    if do_proj:
        in_specs.append(pl.BlockSpec((H, OUT), lambda m: (0, 0)))
        args.append(wo)
        in_specs.append(pl.BlockSpec((1, OUT), lambda m: (0, 0)))
        args.append(bo)
        out_spec = pl.BlockSpec((BM, OUT), lambda m: (m, 0))
        out_shape = jax.ShapeDtypeStruct((N, OUT), jnp.float32)
    else:
        out_spec = pl.BlockSpec((KC, BM, CW), lambda m: (0, m, 0))
        out_shape = jax.ShapeDtypeStruct((KC, N, CW), jnp.float32)
    return pl.pallas_call(
        _make_mix_body(n_rel, do_proj, relu_col),
        grid=(N // BM,),
        in_specs=in_specs,
        out_specs=out_spec,
        out_shape=out_shape,
    )(*args)


def _segsum_body(xc, srcidx, dstidx, zacc, s_out,
                 sidx_v, didx_v, rows_v, acc_sh, sem):
    cid = lax.axis_index("c")
    sid = lax.axis_index("s")
    pltpu.sync_copy(srcidx.at[sid], sidx_v)
    pltpu.sync_copy(dstidx.at[sid], didx_v)
    for ccl in range(2):
        cc = cid * 2 + ccl
        pltpu.sync_copy(zacc, acc_sh.at[pl.ds(sid * ZR, ZR)])
        plsc.subcore_barrier()

        def body(j, carry):
            pltpu.async_copy(xc.at[cc].at[sidx_v.at[j]], rows_v, sem).wait()
            pltpu.sync_copy(rows_v, acc_sh.at[didx_v.at[j]], add=True)
            return carry
        lax.fori_loop(0, NT, body, 0)
        plsc.subcore_barrier()

        pltpu.sync_copy(acc_sh.at[pl.ds(sid * WRS, WRN)],
                        s_out.at[cc, pl.ds(sid * WRS, WRN)])
        plsc.subcore_barrier()


@functools.cache
def _segsum():
    return pl.kernel(
        _segsum_body,
        out_type=jax.ShapeDtypeStruct((KC, N, CW), jnp.float32),
        mesh=plsc.VectorSubcoreMesh(core_axis_name="c", subcore_axis_name="s",
                                    num_cores=2, num_subcores=SUB),
        scratch_types=[
            pltpu.VMEM((NT, TIL), jnp.int32),
            pltpu.VMEM((NT, TIL), jnp.int32),
            pltpu.VMEM((TIL, CW), jnp.float32),
            pltpu.VMEM_SHARED((ACC, CW), jnp.float32),
            pltpu.SemaphoreType.DMA,
        ],
    )


def _count_body(dst_bb, dst_bg, dst_gb, zacc, onec, cbb_out, cbg_out, cgb_out,
                didx_v, ones_v, acc_sh):
    cid = lax.axis_index("c")
    sid = lax.axis_index("s")
    pltpu.sync_copy(onec, ones_v)
    for rnd in range(2):
        if rnd == 0:
            @pl.when(cid == 0)
            def _():
                pltpu.sync_copy(dst_bb.at[sid], didx_v)

            @pl.when(cid == 1)
            def _():
                pltpu.sync_copy(dst_gb.at[sid], didx_v)
        else:
            @pl.when(cid == 0)
            def _():
                pltpu.sync_copy(dst_bg.at[sid], didx_v)

        pltpu.sync_copy(zacc, acc_sh.at[pl.ds(sid * ZR, ZR)])
        plsc.subcore_barrier()

        active = (cid == 0) if rnd == 1 else (cid < 2)

        @pl.when(active)
        def _():
            def body(j, carry):
                pltpu.sync_copy(ones_v, acc_sh.at[didx_v.at[j]], add=True)
                return carry
            lax.fori_loop(0, NT, body, 0)
        plsc.subcore_barrier()

        if rnd == 0:
            @pl.when(cid == 0)
            def _():
                pltpu.sync_copy(acc_sh.at[pl.ds(sid * WRS, WRN)],
                                cbb_out.at[pl.ds(sid * WRS, WRN)])

            @pl.when(cid == 1)
            def _():
                pltpu.sync_copy(acc_sh.at[pl.ds(sid * WRS, WRN)],
                                cgb_out.at[pl.ds(sid * WRS, WRN)])
        else:
            @pl.when(cid == 0)
            def _():
                pltpu.sync_copy(acc_sh.at[pl.ds(sid * WRS, WRN)],
                                cbg_out.at[pl.ds(sid * WRS, WRN)])
        plsc.subcore_barrier()


@functools.cache
def _count():
    return pl.kernel(
        _count_body,
        out_type=(jax.ShapeDtypeStruct((N, CW), jnp.float32),
                  jax.ShapeDtypeStruct((N, CW), jnp.float32),
                  jax.ShapeDtypeStruct((N, CW), jnp.float32)),
        mesh=plsc.VectorSubcoreMesh(core_axis_name="c", subcore_axis_name="s",
                                    num_cores=2, num_subcores=SUB),
        scratch_types=[
            pltpu.VMEM((NT, TIL), jnp.int32),
            pltpu.VMEM((TIL, CW), jnp.float32),
            pltpu.VMEM_SHARED((ACC, CW), jnp.float32),
        ],
    )


def _pad_idx(ei):
    src = ei[0].astype(jnp.int32).reshape(SUB, EPW)
    dst = ei[1].astype(jnp.int32).reshape(SUB, EPW)
    pads = ((0, 0), (0, EPAD - EPW))
    src = jnp.pad(src, pads, constant_values=0)
    dst = jnp.pad(dst, pads, constant_values=DEAD)
    return src.reshape(SUB, NT, TIL), dst.reshape(SUB, NT, TIL)


def kernel(x_bus, x_generator, ei_bb, ei_bg, ei_gb,
           lin_bus_W, lin_bus_b, lin_gen_W, lin_gen_b,
           c0_bb_Wl, c0_bb_bl, c0_bb_Wr, c0_bg_Wl, c0_bg_bl, c0_bg_Wr,
           c0_gb_Wl, c0_gb_bl, c0_gb_Wr,
           c1_bb_Wl, c1_bb_bl, c1_bb_Wr, c1_bg_Wl, c1_bg_bl, c1_bg_Wr,
           c1_gb_Wl, c1_gb_bl, c1_gb_Wr,
           out_bus_W, out_bus_b, out_gen_W, out_gen_b):
    f32 = jnp.float32

    src_bb, dst_bb = _pad_idx(ei_bb)
    src_bg, dst_bg = _pad_idx(ei_bg)
    src_gb, dst_gb = _pad_idx(ei_gb)
    zacc = jnp.zeros((ZR, CW), f32)
    onec = jnp.ones((TIL, CW), f32)

    wcat_b0 = jnp.concatenate([c0_bb_Wl.T, c0_gb_Wl.T, (c0_bb_Wr + c0_gb_Wr).T], axis=0)
    bias_b0 = (c0_bb_bl + c0_gb_bl).reshape(1, H)
    wcat_g0 = jnp.concatenate([c0_bg_Wl.T, c0_bg_Wr.T], axis=0)
    bias_g0 = c0_bg_bl.reshape(1, H)
    wcat_b1 = jnp.concatenate([c1_bb_Wl.T, c1_gb_Wl.T, (c1_bb_Wr + c1_gb_Wr).T], axis=0)
    bias_b1 = (c1_bb_bl + c1_gb_bl).reshape(1, H)
    wcat_g1 = jnp.concatenate([c1_bg_Wl.T, c1_bg_Wr.T], axis=0)
    bias_g1 = c1_bg_bl.reshape(1, H)

    xb0 = _proj(x_bus, lin_bus_W.T, lin_bus_b.reshape(1, H))
    xg0 = _proj(x_generator, lin_gen_W.T, lin_gen_b.reshape(1, H))

    cnt_bb, cnt_bg, cnt_gb = _count()(dst_bb, dst_bg, dst_gb, zacc, onec)
    cbb = cnt_bb[:, :1]
    cgb = cnt_gb[:, :1]
    cbg = cnt_bg[:, :1]

    seg = _segsum()
    sbb0 = seg(xb0, src_bb, dst_bb, zacc)
    sgb0 = seg(xg0, src_gb, dst_gb, zacc)
    sbg0 = seg(xb0, src_bg, dst_bg, zacc)

    xb1 = _mix(2, False, None, [sbb0, sgb0], [cbb, cgb], xb0, wcat_b0, bias_b0)
    xg1 = _mix(1, False, None, [sbg0], [cbg], xg0, wcat_g0, bias_g0)

    sbb1 = seg(xb1, src_bb, dst_bb, zacc)
    sgb1 = seg(xg1, src_gb, dst_gb, zacc)
    sbg1 = seg(xb1, src_bg, dst_bg, zacc)

    bus_out = _mix(2, True, 1, [sbb1, sgb1], [cbb, cgb], xb1, wcat_b1, bias_b1,
                   out_bus_W.T, out_bus_b.reshape(1, OUT))
    gen_out = _mix(1, True, None, [sbg1], [cbg], xg1, wcat_g1, bias_g1,
                   out_gen_W.T, out_gen_b.reshape(1, OUT))
    return (bus_out, gen_out)
